# baseline (speedup 1.0000x reference)
import functools

import jax
import jax.numpy as jnp
from jax.experimental import pallas as pl
from jax.experimental.pallas import tpu as pltpu


def _round_up(x: int, m: int) -> int:
    return ((x + m - 1) // m) * m


def _pool_spatial_major_kernel(x_ref, o_ref, *, inv_n: float):
    x = x_ref[...].astype(jnp.float32)
    o_ref[...] = (jnp.sum(x, axis=0) * inv_n).astype(o_ref.dtype)


def _pool_rows_kernel(x_ref, o_ref, *, inv_n: float):
    x = x_ref[...].astype(jnp.float32)
    o_ref[...] = (jnp.sum(x, axis=-1, keepdims=True) * inv_n).astype(o_ref.dtype)


def kernel(x):
    b, c, h, w = x.shape
    spatial = h * w
    inv_n = 1.0 / float(spatial)

    if b % 8 == 0 and c % 128 == 0:
        xt = jax.lax.transpose(x, (2, 3, 0, 1)).reshape(spatial, b, c)
        tb = 8
        tc = c // 2 if c % 256 == 0 else c
        return pl.pallas_call(
            functools.partial(_pool_spatial_major_kernel, inv_n=inv_n),
            out_shape=jax.ShapeDtypeStruct((b, c), x.dtype),
            grid=(b // tb, c // tc),
            in_specs=[pl.BlockSpec((spatial, tb, tc), lambda i, j: (0, i, j))],
            out_specs=pl.BlockSpec((tb, tc), lambda i, j: (i, j)),
            compiler_params=pltpu.CompilerParams(
                dimension_semantics=("parallel", "parallel"),
                vmem_limit_bytes=64 * 1024 * 1024,
            ),
        )(xt)

    rows = b * c
    xf = x.reshape(rows, spatial)
    tm = min(4096, _round_up(rows, 8))
    r_pad = _round_up(rows, tm)
    if r_pad != rows:
        xf = jnp.pad(xf, ((0, r_pad - rows), (0, 0)))
    out = pl.pallas_call(
        functools.partial(_pool_rows_kernel, inv_n=inv_n),
        out_shape=jax.ShapeDtypeStruct((r_pad, 1), x.dtype),
        grid=(r_pad // tm,),
        in_specs=[pl.BlockSpec((tm, spatial), lambda i: (i, 0))],
        out_specs=pl.BlockSpec((tm, 1), lambda i: (i, 0)),
        compiler_params=pltpu.CompilerParams(
            dimension_semantics=("parallel",),
            vmem_limit_bytes=64 * 1024 * 1024,
        ),
    )(xf)
    return out[:rows, 0].reshape(b, c)


# speedup vs baseline: 1.1822x; 1.1822x over previous
import functools

import jax
import jax.numpy as jnp
from jax.experimental import pallas as pl
from jax.experimental.pallas import tpu as pltpu


def _round_up(x: int, m: int) -> int:
    return ((x + m - 1) // m) * m


def _pool_spatial_major_kernel(x_ref, o_ref, *, inv_n: float):
    x = x_ref[...].astype(jnp.float32)
    o_ref[...] = (jnp.sum(x, axis=0) * inv_n).astype(o_ref.dtype)


def _pool_rows_kernel(x_ref, o_ref, *, inv_n: float):
    x = x_ref[...].astype(jnp.float32)
    o_ref[...] = (jnp.sum(x, axis=-1, keepdims=True) * inv_n).astype(o_ref.dtype)


def kernel(x):
    b, c, h, w = x.shape
    spatial = h * w
    inv_n = 1.0 / float(spatial)

    if b % 8 == 0 and c % 128 == 0:
        xt = jax.lax.transpose(x, (2, 3, 0, 1)).reshape(spatial, b, c)
        tb = 8
        return pl.pallas_call(
            functools.partial(_pool_spatial_major_kernel, inv_n=inv_n),
            out_shape=jax.ShapeDtypeStruct((b, c), x.dtype),
            grid=(b // tb,),
            in_specs=[pl.BlockSpec((spatial, tb, c), lambda i: (0, i, 0))],
            out_specs=pl.BlockSpec((tb, c), lambda i: (i, 0)),
            compiler_params=pltpu.CompilerParams(
                dimension_semantics=("parallel",),
                vmem_limit_bytes=64 * 1024 * 1024,
            ),
        )(xt)

    rows = b * c
    xf = x.reshape(rows, spatial)
    tm = min(4096, _round_up(rows, 8))
    r_pad = _round_up(rows, tm)
    if r_pad != rows:
        xf = jnp.pad(xf, ((0, r_pad - rows), (0, 0)))
    out = pl.pallas_call(
        functools.partial(_pool_rows_kernel, inv_n=inv_n),
        out_shape=jax.ShapeDtypeStruct((r_pad, 1), x.dtype),
        grid=(r_pad // tm,),
        in_specs=[pl.BlockSpec((tm, spatial), lambda i: (i, 0))],
        out_specs=pl.BlockSpec((tm, 1), lambda i: (i, 0)),
        compiler_params=pltpu.CompilerParams(
            dimension_semantics=("parallel",),
            vmem_limit_bytes=64 * 1024 * 1024,
        ),
    )(xf)
    return out[:rows, 0].reshape(b, c)
